# baseline (speedup 1.0000x reference)
import jax
import jax.numpy as jnp
from jax import lax
from jax.experimental import pallas as pl
from jax.experimental.pallas import tpu as pltpu
from jax.experimental.pallas import tpu_sc as plsc

N_USERS = 25000
N = 50000
NP = 51200
DIM = 64
PD = 16
NPL = 4
E = 800000
CH = 128
CPT = 400
QC = 40
NQ = CPT // QC
EPAD = 16 * CPT * CH
NCHUNK = EPAD // CH
NTILES = 16
STRIPE = NP // NTILES
WCH = 128
SN = STRIPE // WCH


def _body(emb, cols2, rows2, vals2, pooled,
          scr_a, scr_b, scr_e, srcs, acc,
          g0, g1, g2, g3, mc, mr, mv, wb0, wb1,
          sg0, sg1, sg2, sg3, ss0, ss1, ss2, ss3, wo0, wo1, fi0, fi1):
    c = lax.axis_index("c")
    sid = lax.axis_index("s")
    gbufs = (g0, g1, g2, g3)
    gsems = (sg0, sg1, sg2, sg3)
    ssems = (ss0, ss1, ss2, ss3)
    fsems = (fi0, fi1)
    wbufs = (wb0, wb1)
    wsems = (wo0, wo1)
    zero16 = jnp.zeros((16,), jnp.float32)
    base = sid * STRIPE

    def zero_g3():
        @pl.loop(0, WCH)
        def _zz(r):
            g3[r, 0:16] = zero16

    def fire_gather(q, b, hsrc, plane):
        if b != 3:
            pltpu.async_copy(srcs.at[mc.at[q]], gbufs[b], gsems[b])
        else:
            pltpu.async_copy(hsrc.at[pl.ds(plane * NP, NP)].at[mc.at[q]],
                             gbufs[b], gsems[b])

    def wait_gather(b):
        pltpu.make_async_copy(srcs.at[mc.at[0]], gbufs[b], gsems[b]).wait()

    def fire_scatter(q, b):
        pltpu.async_copy(gbufs[b], acc.at[mr.at[q]], ssems[b], add=True)

    def wait_scatter(b):
        pltpu.make_async_copy(gbufs[b], acc.at[mr.at[0]], ssems[b]).wait()

    def scale(q, b):
        g = gbufs[b]

        @pl.loop(0, CH, step=16)
        def _s(e):
            v16 = mv[q, pl.ds(e, 16)]
            for i in range(16):
                g[e + i, 0:16] = g[e + i, 0:16] * v16[i]

    def stage(plane):
        zero_g3()

        @pl.loop(0, SN)
        def _st(k):
            off = base + k * WCH
            pltpu.sync_copy(emb.at[pl.ds(off, WCH), pl.ds(plane * PD, PD)],
                            wb0)
            pltpu.sync_copy(wb0, srcs.at[pl.ds(off, WCH)])
            pltpu.sync_copy(wb0, scr_e.at[pl.ds(plane * NP + off, WCH)])
            pltpu.sync_copy(g3, acc.at[pl.ds(off, WCH)])

    def edge_phase(hsrc, plane):
        @pl.loop(0, NQ)
        def _q(qtr):
            mbase = sid * CPT + qtr * QC
            pltpu.sync_copy(cols2.at[pl.ds(mbase, QC)], mc)
            pltpu.sync_copy(rows2.at[pl.ds(mbase, QC)], mr)
            pltpu.sync_copy(vals2.at[pl.ds(mbase, QC)], mv)
            fire_gather(0, 0, hsrc, plane)
            fire_gather(1, 1, hsrc, plane)
            fire_gather(2, 2, hsrc, plane)
            wait_gather(0)
            scale(0, 0)
            fire_scatter(0, 0)
            fire_gather(3, 3, hsrc, plane)
            wait_gather(1)
            scale(1, 1)
            fire_scatter(1, 1)
            for m in (2, 3):
                wait_scatter(m - 2)
                fire_gather(m + 2, (m + 2) % 4, hsrc, plane)
                wait_gather(m)
                scale(m, m)
                fire_scatter(m, m)

            @pl.loop(4, QC - 4, step=4)
            def _p(p):
                for m in range(4):
                    q = p + m
                    bn = (m + 2) % 4
                    wait_scatter(bn)
                    fire_gather(q + 2, bn, hsrc, plane)
                    wait_gather(m)
                    scale(q, m)
                    fire_scatter(q, m)

            for m in range(4):
                q = QC - 4 + m
                bn = (m + 2) % 4
                wait_scatter(bn)
                if m < 2:
                    fire_gather(q + 2, bn, hsrc, plane)
                wait_gather(m)
                scale(q, m)
                fire_scatter(q, m)
            wait_scatter(2)
            wait_scatter(3)

    def writeback(dst, plane):
        pbase = plane * NP + base
        zero_g3()

        def sub(k, m):
            off = base + k * WCH
            pltpu.sync_copy(acc.at[pl.ds(off, WCH)], wbufs[m])
            pltpu.sync_copy(wbufs[m], srcs.at[pl.ds(off, WCH)])
            pltpu.sync_copy(g3, acc.at[pl.ds(off, WCH)])
            pltpu.async_copy(wbufs[m], dst.at[pl.ds(pbase + k * WCH, WCH)],
                             wsems[m])

        def wait_out(m):
            pltpu.make_async_copy(wbufs[m], dst.at[pl.ds(pbase, WCH)],
                                  wsems[m]).wait()

        sub(0, 0)
        sub(1, 1)

        @pl.loop(2, SN - 1, step=2)
        def _w(k):
            for m in range(2):
                wait_out(m)
                sub(k + m, m)

        wait_out(0)
        sub(SN - 1, 0)
        wait_out(0)
        wait_out(1)

    def final_pass(plane):
        pbase = plane * NP + base

        def sub(k, m):
            w = wbufs[m]
            off = base + k * WCH
            pltpu.sync_copy(acc.at[pl.ds(off, WCH)], w)
            pltpu.async_copy(
                emb.at[pl.ds(off, WCH), pl.ds(plane * PD, PD)], g0, fi0)
            pltpu.async_copy(scr_a.at[pl.ds(pbase + k * WCH, WCH)], g1, fi1)
            for (fm, fbuf, nxt) in ((0, g0, scr_b), (1, g1, None)):
                pltpu.make_async_copy(scr_a.at[pl.ds(0, WCH)], fbuf,
                                      fsems[fm]).wait()

                @pl.loop(0, WCH, unroll=4)
                def _f(r):
                    w[r, 0:16] = w[r, 0:16] + fbuf[r, 0:16]

                if nxt is not None:
                    pltpu.async_copy(nxt.at[pl.ds(pbase + k * WCH, WCH)],
                                     fbuf, fsems[fm])
            pltpu.make_async_copy(scr_a.at[pl.ds(0, WCH)], g0, fi0).wait()

            @pl.loop(0, WCH, unroll=4)
            def _f3(r):
                w[r, 0:16] = w[r, 0:16] + g0[r, 0:16]

            pltpu.async_copy(
                w, pooled.at[pl.ds(off, WCH), pl.ds(plane * PD, PD)],
                wsems[m])

        def wait_out(m):
            pltpu.make_async_copy(
                wbufs[m], pooled.at[pl.ds(base, WCH), pl.ds(0, PD)],
                wsems[m]).wait()

        sub(0, 0)
        sub(1, 1)

        @pl.loop(2, SN - 1, step=2)
        def _w(k):
            for m in range(2):
                wait_out(m)
                sub(k + m, m)

        wait_out(0)
        sub(SN - 1, 0)
        wait_out(0)
        wait_out(1)

    @pl.loop(0, 2)
    def _planes(t):
        plane = 2 * c + t
        stage(plane)
        plsc.subcore_barrier()
        edge_phase(scr_e, plane)
        plsc.subcore_barrier()
        writeback(scr_a, plane)
        plsc.subcore_barrier()
        edge_phase(scr_a, plane)
        plsc.subcore_barrier()
        writeback(scr_b, plane)
        plsc.subcore_barrier()
        edge_phase(scr_b, plane)
        plsc.subcore_barrier()
        final_pass(plane)
        plsc.subcore_barrier()


@jax.jit
def _run(emb, cols2, rows2, vals2):
    mesh = plsc.VectorSubcoreMesh(core_axis_name="c", subcore_axis_name="s")
    f = pl.kernel(
        _body,
        out_type=jax.ShapeDtypeStruct((NP, DIM), jnp.float32),
        mesh=mesh,
        scratch_types=[
            pltpu.HBM((NPL * NP, PD), jnp.float32),
            pltpu.HBM((NPL * NP, PD), jnp.float32),
            pltpu.HBM((NPL * NP, PD), jnp.float32),
            pltpu.VMEM_SHARED((NP, PD), jnp.float32),
            pltpu.VMEM_SHARED((NP, PD), jnp.float32),
            pltpu.VMEM((CH, PD), jnp.float32),
            pltpu.VMEM((CH, PD), jnp.float32),
            pltpu.VMEM((CH, PD), jnp.float32),
            pltpu.VMEM((CH, PD), jnp.float32),
            pltpu.VMEM((QC, CH), jnp.int32),
            pltpu.VMEM((QC, CH), jnp.int32),
            pltpu.VMEM((QC, CH), jnp.float32),
            pltpu.VMEM((WCH, PD), jnp.float32),
            pltpu.VMEM((WCH, PD), jnp.float32),
        ] + [pltpu.SemaphoreType.DMA] * 12,
        compiler_params=pltpu.CompilerParams(
            needs_layout_passes=False, use_tc_tiling_on_sc=False),
    )
    return f(emb, cols2, rows2, vals2)


def kernel(user_embed, item_embed, edge_rows, edge_cols, edge_vals,
           batch, mess_dropout, edge_dropout):
    all_embed = jnp.concatenate([user_embed, item_embed], axis=0)
    emb = jnp.concatenate(
        [all_embed, jnp.zeros((NP - N, DIM), jnp.float32)], axis=0)
    pad = EPAD - E
    pr = jnp.arange(pad, dtype=jnp.int32) % N
    cols_p = jnp.concatenate([edge_cols.astype(jnp.int32), pr])
    rows_p = jnp.concatenate([edge_rows.astype(jnp.int32), pr])
    vals_p = jnp.concatenate([edge_vals, jnp.zeros((pad,), jnp.float32)])
    cols2 = cols_p.reshape(NCHUNK, CH)
    rows2 = rows_p.reshape(NCHUNK, CH)
    vals2 = vals_p.reshape(NCHUNK, CH)
    pooled = _run(emb, cols2, rows2, vals2)
    return (pooled[:N_USERS], pooled[N_USERS:N])


# speedup vs baseline: 1.2138x; 1.2138x over previous
import jax
import jax.numpy as jnp
from jax import lax
from jax.experimental import pallas as pl
from jax.experimental.pallas import tpu as pltpu
from jax.experimental.pallas import tpu_sc as plsc

N_USERS = 25000
N = 50000
NP = 51200
DIM = 64
PD = 16
NPL = 4
E = 800000
CH = 128
CPT = 400
QC = 40
NQ = CPT // QC
EPAD = 16 * CPT * CH
NCHUNK = EPAD // CH
NTILES = 16
STRIPE = NP // NTILES
WCH = 128
SN = STRIPE // WCH


def _body(emb, cols2, rows2, vals2, pooled,
          scr_a, scr_b, s0, s1,
          g0, g1, g2, g3, mc, mr, mv, wb0, wb1,
          sg0, sg1, sg2, sg3, ss0, ss1, ss2, ss3, wo0, wo1, fi0, fi1):
    c = lax.axis_index("c")
    sid = lax.axis_index("s")
    gbufs = (g0, g1, g2, g3)
    gsems = (sg0, sg1, sg2, sg3)
    ssems = (ss0, ss1, ss2, ss3)
    fsems = (fi0, fi1)
    wbufs = (wb0, wb1)
    wsems = (wo0, wo1)
    zero16 = jnp.zeros((16,), jnp.float32)
    base = sid * STRIPE

    def zero_g3():
        @pl.loop(0, WCH)
        def _zz(r):
            g3[r, 0:16] = zero16

    def fire_gather(srcs, q, b):
        pltpu.async_copy(srcs.at[mc.at[q]], gbufs[b], gsems[b])

    def wait_gather(b):
        pltpu.make_async_copy(s0.at[mc.at[0]], gbufs[b], gsems[b]).wait()

    def fire_scatter(acc, q, b):
        pltpu.async_copy(gbufs[b], acc.at[mr.at[q]], ssems[b], add=True)

    def wait_scatter(b):
        pltpu.make_async_copy(gbufs[b], s1.at[mr.at[0]], ssems[b]).wait()

    def scale(q, b):
        g = gbufs[b]

        @pl.loop(0, CH, step=16)
        def _s(e):
            v16 = mv[q, pl.ds(e, 16)]
            for i in range(16):
                g[e + i, 0:16] = g[e + i, 0:16] * v16[i]

    def stage(plane):
        zero_g3()

        @pl.loop(0, SN)
        def _st(k):
            off = base + k * WCH
            pltpu.sync_copy(emb.at[pl.ds(off, WCH), pl.ds(plane * PD, PD)],
                            wb0)
            pltpu.sync_copy(wb0, s0.at[pl.ds(off, WCH)])
            pltpu.sync_copy(g3, s1.at[pl.ds(off, WCH)])

    def edge_phase(srcs, acc):
        @pl.loop(0, NQ)
        def _q(qtr):
            mbase = sid * CPT + qtr * QC
            pltpu.sync_copy(cols2.at[pl.ds(mbase, QC)], mc)
            pltpu.sync_copy(rows2.at[pl.ds(mbase, QC)], mr)
            pltpu.sync_copy(vals2.at[pl.ds(mbase, QC)], mv)
            fire_gather(srcs, 0, 0)
            fire_gather(srcs, 1, 1)
            fire_gather(srcs, 2, 2)
            wait_gather(0)
            scale(0, 0)
            fire_scatter(acc, 0, 0)
            fire_gather(srcs, 3, 3)
            wait_gather(1)
            scale(1, 1)
            fire_scatter(acc, 1, 1)
            for m in (2, 3):
                wait_scatter(m - 2)
                fire_gather(srcs, m + 2, (m + 2) % 4)
                wait_gather(m)
                scale(m, m)
                fire_scatter(acc, m, m)

            @pl.loop(4, QC - 4, step=4)
            def _p(p):
                for m in range(4):
                    q = p + m
                    bn = (m + 2) % 4
                    wait_scatter(bn)
                    fire_gather(srcs, q + 2, bn)
                    wait_gather(m)
                    scale(q, m)
                    fire_scatter(acc, q, m)

            for m in range(4):
                q = QC - 4 + m
                bn = (m + 2) % 4
                wait_scatter(bn)
                if m < 2:
                    fire_gather(srcs, q + 2, bn)
                wait_gather(m)
                scale(q, m)
                fire_scatter(acc, q, m)
            wait_scatter(2)
            wait_scatter(3)

    def writeback(dst, plane, acc, newacc):
        pbase = plane * NP + base
        zero_g3()

        def sub(k, m):
            off = base + k * WCH
            pltpu.sync_copy(acc.at[pl.ds(off, WCH)], wbufs[m])
            pltpu.sync_copy(g3, newacc.at[pl.ds(off, WCH)])
            pltpu.async_copy(wbufs[m], dst.at[pl.ds(pbase + k * WCH, WCH)],
                             wsems[m])

        def wait_out(m):
            pltpu.make_async_copy(wbufs[m], dst.at[pl.ds(pbase, WCH)],
                                  wsems[m]).wait()

        sub(0, 0)
        sub(1, 1)

        @pl.loop(2, SN - 1, step=2)
        def _w(k):
            for m in range(2):
                wait_out(m)
                sub(k + m, m)

        wait_out(0)
        sub(SN - 1, 0)
        wait_out(0)
        wait_out(1)

    def final_pass(plane, acc):
        pbase = plane * NP + base

        def sub(k, m):
            w = wbufs[m]
            off = base + k * WCH
            pltpu.sync_copy(acc.at[pl.ds(off, WCH)], w)
            pltpu.async_copy(
                emb.at[pl.ds(off, WCH), pl.ds(plane * PD, PD)], g0, fi0)
            pltpu.async_copy(scr_a.at[pl.ds(pbase + k * WCH, WCH)], g1, fi1)
            for (fm, fbuf, nxt) in ((0, g0, scr_b), (1, g1, None)):
                pltpu.make_async_copy(scr_a.at[pl.ds(0, WCH)], fbuf,
                                      fsems[fm]).wait()

                @pl.loop(0, WCH, unroll=4)
                def _f(r):
                    w[r, 0:16] = w[r, 0:16] + fbuf[r, 0:16]

                if nxt is not None:
                    pltpu.async_copy(nxt.at[pl.ds(pbase + k * WCH, WCH)],
                                     fbuf, fsems[fm])
            pltpu.make_async_copy(scr_a.at[pl.ds(0, WCH)], g0, fi0).wait()

            @pl.loop(0, WCH, unroll=4)
            def _f3(r):
                w[r, 0:16] = w[r, 0:16] + g0[r, 0:16]

            pltpu.async_copy(
                w, pooled.at[pl.ds(off, WCH), pl.ds(plane * PD, PD)],
                wsems[m])

        def wait_out(m):
            pltpu.make_async_copy(
                wbufs[m], pooled.at[pl.ds(base, WCH), pl.ds(0, PD)],
                wsems[m]).wait()

        sub(0, 0)
        sub(1, 1)

        @pl.loop(2, SN - 1, step=2)
        def _w(k):
            for m in range(2):
                wait_out(m)
                sub(k + m, m)

        wait_out(0)
        sub(SN - 1, 0)
        wait_out(0)
        wait_out(1)

    @pl.loop(0, 2)
    def _planes(t):
        plane = 2 * c + t
        stage(plane)
        plsc.subcore_barrier()
        edge_phase(s0, s1)
        plsc.subcore_barrier()
        writeback(scr_a, plane, s1, s0)
        plsc.subcore_barrier()
        edge_phase(s1, s0)
        plsc.subcore_barrier()
        writeback(scr_b, plane, s0, s1)
        plsc.subcore_barrier()
        edge_phase(s0, s1)
        plsc.subcore_barrier()
        final_pass(plane, s1)
        plsc.subcore_barrier()


@jax.jit
def _run(emb, cols2, rows2, vals2):
    mesh = plsc.VectorSubcoreMesh(core_axis_name="c", subcore_axis_name="s")
    f = pl.kernel(
        _body,
        out_type=jax.ShapeDtypeStruct((NP, DIM), jnp.float32),
        mesh=mesh,
        scratch_types=[
            pltpu.HBM((NPL * NP, PD), jnp.float32),
            pltpu.HBM((NPL * NP, PD), jnp.float32),
            pltpu.VMEM_SHARED((NP, PD), jnp.float32),
            pltpu.VMEM_SHARED((NP, PD), jnp.float32),
            pltpu.VMEM((CH, PD), jnp.float32),
            pltpu.VMEM((CH, PD), jnp.float32),
            pltpu.VMEM((CH, PD), jnp.float32),
            pltpu.VMEM((CH, PD), jnp.float32),
            pltpu.VMEM((QC, CH), jnp.int32),
            pltpu.VMEM((QC, CH), jnp.int32),
            pltpu.VMEM((QC, CH), jnp.float32),
            pltpu.VMEM((WCH, PD), jnp.float32),
            pltpu.VMEM((WCH, PD), jnp.float32),
        ] + [pltpu.SemaphoreType.DMA] * 12,
        compiler_params=pltpu.CompilerParams(
            needs_layout_passes=False, use_tc_tiling_on_sc=False),
    )
    return f(emb, cols2, rows2, vals2)


def kernel(user_embed, item_embed, edge_rows, edge_cols, edge_vals,
           batch, mess_dropout, edge_dropout):
    all_embed = jnp.concatenate([user_embed, item_embed], axis=0)
    emb = jnp.concatenate(
        [all_embed, jnp.zeros((NP - N, DIM), jnp.float32)], axis=0)
    pad = EPAD - E
    pr = jnp.arange(pad, dtype=jnp.int32) % N
    cols_p = jnp.concatenate([edge_cols.astype(jnp.int32), pr])
    rows_p = jnp.concatenate([edge_rows.astype(jnp.int32), pr])
    vals_p = jnp.concatenate([edge_vals, jnp.zeros((pad,), jnp.float32)])
    cols2 = cols_p.reshape(NCHUNK, CH)
    rows2 = rows_p.reshape(NCHUNK, CH)
    vals2 = vals_p.reshape(NCHUNK, CH)
    pooled = _run(emb, cols2, rows2, vals2)
    return (pooled[:N_USERS], pooled[N_USERS:N])


# speedup vs baseline: 1.2975x; 1.0690x over previous
import jax
import jax.numpy as jnp
from jax import lax
from jax.experimental import pallas as pl
from jax.experimental.pallas import tpu as pltpu
from jax.experimental.pallas import tpu_sc as plsc

N_USERS = 25000
N = 50000
NP = 51200
DIM = 64
PD = 16
NPL = 4
E = 800000
CH = 128
CPT = 400
QC = 40
NQ = CPT // QC
EPAD = 16 * CPT * CH
NCHUNK = EPAD // CH
NTILES = 16
STRIPE = NP // NTILES
WCH = 128
SN = STRIPE // WCH


def _body(emb, cols2, rows2, vals2, pooled,
          scr_a, scr_b, s0, s1,
          g0, g1, g2, g3, mc, mr, mv, wb0, wb1,
          sg0, sg1, sg2, sg3, ss0, ss1, ss2, ss3, wo0, wo1, fi0, fi1):
    c = lax.axis_index("c")
    sid = lax.axis_index("s")
    gbufs = (g0, g1, g2, g3)
    gsems = (sg0, sg1, sg2, sg3)
    ssems = (ss0, ss1, ss2, ss3)
    fsems = (fi0, fi1)
    wbufs = (wb0, wb1)
    wsems = (wo0, wo1)
    zero16 = jnp.zeros((16,), jnp.float32)
    base = sid * STRIPE

    def zero_g3():
        @pl.loop(0, WCH)
        def _zz(r):
            g3[r, 0:16] = zero16

    def fire_gather(srcs, q, b):
        pltpu.async_copy(srcs.at[mc.at[q]], gbufs[b], gsems[b])

    def wait_gather(b):
        pltpu.make_async_copy(s0.at[mc.at[0]], gbufs[b], gsems[b]).wait()

    def fire_scatter(acc, q, b):
        pltpu.async_copy(gbufs[b], acc.at[mr.at[q]], ssems[b], add=True)

    def wait_scatter(b):
        pltpu.make_async_copy(gbufs[b], s1.at[mr.at[0]], ssems[b]).wait()

    def scale(q, b):
        g = gbufs[b]

        @pl.loop(0, CH, step=16)
        def _s(e):
            v16 = mv[q, pl.ds(e, 16)]
            for i in range(16):
                g[e + i, 0:16] = g[e + i, 0:16] * v16[i]

    def stage(plane):
        zero_g3()

        @pl.loop(0, SN)
        def _st(k):
            off = base + k * WCH
            pltpu.sync_copy(emb.at[pl.ds(off, WCH), pl.ds(plane * PD, PD)],
                            wb0)
            pltpu.sync_copy(wb0, s0.at[pl.ds(off, WCH)])
            pltpu.sync_copy(g3, s1.at[pl.ds(off, WCH)])

    def edge_phase(srcs, acc):
        @pl.loop(0, NQ)
        def _q(qtr):
            mbase = sid * CPT + qtr * QC
            pltpu.async_copy(cols2.at[pl.ds(mbase, QC)], mc, fi0)
            pltpu.async_copy(rows2.at[pl.ds(mbase, QC)], mr, fi1)
            pltpu.async_copy(vals2.at[pl.ds(mbase, QC)], mv, wo0)
            pltpu.make_async_copy(cols2.at[pl.ds(0, QC)], mc, fi0).wait()
            pltpu.make_async_copy(rows2.at[pl.ds(0, QC)], mr, fi1).wait()
            pltpu.make_async_copy(vals2.at[pl.ds(0, QC)], mv, wo0).wait()
            fire_gather(srcs, 0, 0)
            fire_gather(srcs, 1, 1)
            fire_gather(srcs, 2, 2)
            wait_gather(0)
            scale(0, 0)
            fire_scatter(acc, 0, 0)
            fire_gather(srcs, 3, 3)
            wait_gather(1)
            scale(1, 1)
            fire_scatter(acc, 1, 1)
            for m in (2, 3):
                wait_scatter(m - 2)
                fire_gather(srcs, m + 2, (m + 2) % 4)
                wait_gather(m)
                scale(m, m)
                fire_scatter(acc, m, m)

            @pl.loop(4, QC - 4, step=4)
            def _p(p):
                for m in range(4):
                    q = p + m
                    bn = (m + 2) % 4
                    wait_scatter(bn)
                    fire_gather(srcs, q + 2, bn)
                    wait_gather(m)
                    scale(q, m)
                    fire_scatter(acc, q, m)

            for m in range(4):
                q = QC - 4 + m
                bn = (m + 2) % 4
                wait_scatter(bn)
                if m < 2:
                    fire_gather(srcs, q + 2, bn)
                wait_gather(m)
                scale(q, m)
                fire_scatter(acc, q, m)
            wait_scatter(2)
            wait_scatter(3)

    def writeback(dst, plane, acc, newacc):
        pbase = plane * NP + base
        zero_g3()

        def sub(k, m):
            off = base + k * WCH
            pltpu.sync_copy(acc.at[pl.ds(off, WCH)], wbufs[m])
            pltpu.sync_copy(g3, newacc.at[pl.ds(off, WCH)])
            pltpu.async_copy(wbufs[m], dst.at[pl.ds(pbase + k * WCH, WCH)],
                             wsems[m])

        def wait_out(m):
            pltpu.make_async_copy(wbufs[m], dst.at[pl.ds(pbase, WCH)],
                                  wsems[m]).wait()

        sub(0, 0)
        sub(1, 1)

        @pl.loop(2, SN - 1, step=2)
        def _w(k):
            for m in range(2):
                wait_out(m)
                sub(k + m, m)

        wait_out(0)
        sub(SN - 1, 0)
        wait_out(0)
        wait_out(1)

    def final_pass(plane, acc):
        pbase = plane * NP + base

        def sub(k, m):
            w = wbufs[m]
            off = base + k * WCH
            pltpu.sync_copy(acc.at[pl.ds(off, WCH)], w)
            pltpu.async_copy(
                emb.at[pl.ds(off, WCH), pl.ds(plane * PD, PD)], g0, fi0)
            pltpu.async_copy(scr_a.at[pl.ds(pbase + k * WCH, WCH)], g1, fi1)
            for (fm, fbuf, nxt) in ((0, g0, scr_b), (1, g1, None)):
                pltpu.make_async_copy(scr_a.at[pl.ds(0, WCH)], fbuf,
                                      fsems[fm]).wait()

                @pl.loop(0, WCH, unroll=4)
                def _f(r):
                    w[r, 0:16] = w[r, 0:16] + fbuf[r, 0:16]

                if nxt is not None:
                    pltpu.async_copy(nxt.at[pl.ds(pbase + k * WCH, WCH)],
                                     fbuf, fsems[fm])
            pltpu.make_async_copy(scr_a.at[pl.ds(0, WCH)], g0, fi0).wait()

            @pl.loop(0, WCH, unroll=4)
            def _f3(r):
                w[r, 0:16] = w[r, 0:16] + g0[r, 0:16]

            pltpu.async_copy(
                w, pooled.at[pl.ds(off, WCH), pl.ds(plane * PD, PD)],
                wsems[m])

        def wait_out(m):
            pltpu.make_async_copy(
                wbufs[m], pooled.at[pl.ds(base, WCH), pl.ds(0, PD)],
                wsems[m]).wait()

        sub(0, 0)
        sub(1, 1)

        @pl.loop(2, SN - 1, step=2)
        def _w(k):
            for m in range(2):
                wait_out(m)
                sub(k + m, m)

        wait_out(0)
        sub(SN - 1, 0)
        wait_out(0)
        wait_out(1)

    @pl.loop(0, 2)
    def _planes(t):
        plane = 2 * c + t
        stage(plane)
        plsc.subcore_barrier()
        edge_phase(s0, s1)
        plsc.subcore_barrier()
        writeback(scr_a, plane, s1, s0)
        plsc.subcore_barrier()
        edge_phase(s1, s0)
        plsc.subcore_barrier()
        writeback(scr_b, plane, s0, s1)
        plsc.subcore_barrier()
        edge_phase(s0, s1)
        plsc.subcore_barrier()
        final_pass(plane, s1)
        plsc.subcore_barrier()


@jax.jit
def _run(emb, cols2, rows2, vals2):
    mesh = plsc.VectorSubcoreMesh(core_axis_name="c", subcore_axis_name="s")
    f = pl.kernel(
        _body,
        out_type=jax.ShapeDtypeStruct((NP, DIM), jnp.float32),
        mesh=mesh,
        scratch_types=[
            pltpu.HBM((NPL * NP, PD), jnp.float32),
            pltpu.HBM((NPL * NP, PD), jnp.float32),
            pltpu.VMEM_SHARED((NP, PD), jnp.float32),
            pltpu.VMEM_SHARED((NP, PD), jnp.float32),
            pltpu.VMEM((CH, PD), jnp.float32),
            pltpu.VMEM((CH, PD), jnp.float32),
            pltpu.VMEM((CH, PD), jnp.float32),
            pltpu.VMEM((CH, PD), jnp.float32),
            pltpu.VMEM((QC, CH), jnp.int32),
            pltpu.VMEM((QC, CH), jnp.int32),
            pltpu.VMEM((QC, CH), jnp.float32),
            pltpu.VMEM((WCH, PD), jnp.float32),
            pltpu.VMEM((WCH, PD), jnp.float32),
        ] + [pltpu.SemaphoreType.DMA] * 12,
        compiler_params=pltpu.CompilerParams(
            needs_layout_passes=False, use_tc_tiling_on_sc=False),
    )
    return f(emb, cols2, rows2, vals2)


def kernel(user_embed, item_embed, edge_rows, edge_cols, edge_vals,
           batch, mess_dropout, edge_dropout):
    all_embed = jnp.concatenate([user_embed, item_embed], axis=0)
    emb = jnp.concatenate(
        [all_embed, jnp.zeros((NP - N, DIM), jnp.float32)], axis=0)
    pad = EPAD - E
    pr = jnp.arange(pad, dtype=jnp.int32) % N
    cols_p = jnp.concatenate([edge_cols.astype(jnp.int32), pr])
    rows_p = jnp.concatenate([edge_rows.astype(jnp.int32), pr])
    vals_p = jnp.concatenate([edge_vals, jnp.zeros((pad,), jnp.float32)])
    cols2 = cols_p.reshape(NCHUNK, CH)
    rows2 = rows_p.reshape(NCHUNK, CH)
    vals2 = vals_p.reshape(NCHUNK, CH)
    pooled = _run(emb, cols2, rows2, vals2)
    return (pooled[:N_USERS], pooled[N_USERS:N])


# speedup vs baseline: 1.2978x; 1.0002x over previous
import jax
import jax.numpy as jnp
from jax import lax
from jax.experimental import pallas as pl
from jax.experimental.pallas import tpu as pltpu
from jax.experimental.pallas import tpu_sc as plsc

N_USERS = 25000
N = 50000
NP = 51200
DIM = 64
PD = 16
NPL = 4
E = 800000
CH = 128
CPT = 400
QC = 40
NQ = CPT // QC
EPAD = 16 * CPT * CH
NCHUNK = EPAD // CH
NTILES = 16
STRIPE = NP // NTILES
WCH = 128
SN = STRIPE // WCH


def _body(emb, cols2, rows2, vals2, pooled,
          scr_a, scr_b, s0, s1,
          g0, g1, g2, g3, mc, mr, mv, wb0, wb1,
          sg0, sg1, sg2, sg3, ss0, ss1, ss2, ss3, wo0, wo1, fi0, fi1,
          mm0, mm1, mm2):
    c = lax.axis_index("c")
    sid = lax.axis_index("s")
    gbufs = (g0, g1, g2, g3)
    gsems = (sg0, sg1, sg2, sg3)
    ssems = (ss0, ss1, ss2, ss3)
    fsems = (fi0, fi1)
    wbufs = (wb0, wb1)
    wsems = (wo0, wo1)
    zero16 = jnp.zeros((16,), jnp.float32)
    base = sid * STRIPE

    def zero_g3():
        @pl.loop(0, WCH)
        def _zz(r):
            g3[r, 0:16] = zero16

    def fire_gather(srcs, q, b):
        pltpu.async_copy(srcs.at[mc.at[q]], gbufs[b], gsems[b])

    def wait_gather(b):
        pltpu.make_async_copy(s0.at[mc.at[0]], gbufs[b], gsems[b]).wait()

    def fire_scatter(acc, q, b):
        pltpu.async_copy(gbufs[b], acc.at[mr.at[q]], ssems[b], add=True)

    def wait_scatter(b):
        pltpu.make_async_copy(gbufs[b], s1.at[mr.at[0]], ssems[b]).wait()

    def scale(q, b):
        g = gbufs[b]

        @pl.loop(0, CH, step=16)
        def _s(e):
            v16 = mv[q, pl.ds(e, 16)]
            for i in range(16):
                g[e + i, 0:16] = g[e + i, 0:16] * v16[i]

    def stage(plane):
        zero_g3()

        @pl.loop(0, SN)
        def _st(k):
            off = base + k * WCH
            pltpu.sync_copy(emb.at[pl.ds(off, WCH), pl.ds(plane * PD, PD)],
                            wb0)
            pltpu.sync_copy(wb0, s0.at[pl.ds(off, WCH)])
            pltpu.sync_copy(g3, s1.at[pl.ds(off, WCH)])

    def edge_phase(srcs, acc):
        @pl.loop(0, NQ)
        def _q(qtr):
            mbase = sid * CPT + qtr * QC
            pltpu.async_copy(cols2.at[pl.ds(mbase, QC)], mc, mm0)
            pltpu.async_copy(rows2.at[pl.ds(mbase, QC)], mr, mm1)
            pltpu.async_copy(vals2.at[pl.ds(mbase, QC)], mv, mm2)
            pltpu.make_async_copy(cols2.at[pl.ds(0, QC)], mc, mm0).wait()
            pltpu.make_async_copy(rows2.at[pl.ds(0, QC)], mr, mm1).wait()
            pltpu.make_async_copy(vals2.at[pl.ds(0, QC)], mv, mm2).wait()
            fire_gather(srcs, 0, 0)
            fire_gather(srcs, 1, 1)
            fire_gather(srcs, 2, 2)
            wait_gather(0)
            scale(0, 0)
            fire_scatter(acc, 0, 0)
            fire_gather(srcs, 3, 3)
            wait_gather(1)
            scale(1, 1)
            fire_scatter(acc, 1, 1)
            for m in (2, 3):
                wait_scatter(m - 2)
                fire_gather(srcs, m + 2, (m + 2) % 4)
                wait_gather(m)
                scale(m, m)
                fire_scatter(acc, m, m)

            @pl.loop(4, QC - 4, step=4)
            def _p(p):
                for m in range(4):
                    q = p + m
                    bn = (m + 2) % 4
                    wait_scatter(bn)
                    fire_gather(srcs, q + 2, bn)
                    wait_gather(m)
                    scale(q, m)
                    fire_scatter(acc, q, m)

            for m in range(4):
                q = QC - 4 + m
                bn = (m + 2) % 4
                wait_scatter(bn)
                if m < 2:
                    fire_gather(srcs, q + 2, bn)
                wait_gather(m)
                scale(q, m)
                fire_scatter(acc, q, m)
            wait_scatter(2)
            wait_scatter(3)

    def writeback(dst, plane, acc, newacc):
        pbase = plane * NP + base
        zero_g3()

        def sub(k, m):
            off = base + k * WCH
            pltpu.sync_copy(acc.at[pl.ds(off, WCH)], wbufs[m])
            pltpu.sync_copy(g3, newacc.at[pl.ds(off, WCH)])
            pltpu.async_copy(wbufs[m], dst.at[pl.ds(pbase + k * WCH, WCH)],
                             wsems[m])

        def wait_out(m):
            pltpu.make_async_copy(wbufs[m], dst.at[pl.ds(pbase, WCH)],
                                  wsems[m]).wait()

        sub(0, 0)
        sub(1, 1)

        @pl.loop(2, SN - 1, step=2)
        def _w(k):
            for m in range(2):
                wait_out(m)
                sub(k + m, m)

        wait_out(0)
        sub(SN - 1, 0)
        wait_out(0)
        wait_out(1)

    def final_pass(plane, acc):
        pbase = plane * NP + base

        def sub(k, m):
            w = wbufs[m]
            off = base + k * WCH
            pltpu.sync_copy(acc.at[pl.ds(off, WCH)], w)
            pltpu.async_copy(
                emb.at[pl.ds(off, WCH), pl.ds(plane * PD, PD)], g0, fi0)
            pltpu.async_copy(scr_a.at[pl.ds(pbase + k * WCH, WCH)], g1, fi1)
            for (fm, fbuf, nxt) in ((0, g0, scr_b), (1, g1, None)):
                pltpu.make_async_copy(scr_a.at[pl.ds(0, WCH)], fbuf,
                                      fsems[fm]).wait()

                @pl.loop(0, WCH, unroll=4)
                def _f(r):
                    w[r, 0:16] = w[r, 0:16] + fbuf[r, 0:16]

                if nxt is not None:
                    pltpu.async_copy(nxt.at[pl.ds(pbase + k * WCH, WCH)],
                                     fbuf, fsems[fm])
            pltpu.make_async_copy(scr_a.at[pl.ds(0, WCH)], g0, fi0).wait()

            @pl.loop(0, WCH, unroll=4)
            def _f3(r):
                w[r, 0:16] = w[r, 0:16] + g0[r, 0:16]

            pltpu.async_copy(
                w, pooled.at[pl.ds(off, WCH), pl.ds(plane * PD, PD)],
                wsems[m])

        def wait_out(m):
            pltpu.make_async_copy(
                wbufs[m], pooled.at[pl.ds(base, WCH), pl.ds(0, PD)],
                wsems[m]).wait()

        sub(0, 0)
        sub(1, 1)

        @pl.loop(2, SN - 1, step=2)
        def _w(k):
            for m in range(2):
                wait_out(m)
                sub(k + m, m)

        wait_out(0)
        sub(SN - 1, 0)
        wait_out(0)
        wait_out(1)

    @pl.loop(0, 2)
    def _planes(t):
        plane = 2 * c + t
        stage(plane)
        plsc.subcore_barrier()
        edge_phase(s0, s1)
        plsc.subcore_barrier()
        writeback(scr_a, plane, s1, s0)
        plsc.subcore_barrier()
        edge_phase(s1, s0)
        plsc.subcore_barrier()
        writeback(scr_b, plane, s0, s1)
        plsc.subcore_barrier()
        edge_phase(s0, s1)
        plsc.subcore_barrier()
        final_pass(plane, s1)
        plsc.subcore_barrier()


@jax.jit
def _run(emb, cols2, rows2, vals2):
    mesh = plsc.VectorSubcoreMesh(core_axis_name="c", subcore_axis_name="s")
    f = pl.kernel(
        _body,
        out_type=jax.ShapeDtypeStruct((NP, DIM), jnp.float32),
        mesh=mesh,
        scratch_types=[
            pltpu.HBM((NPL * NP, PD), jnp.float32),
            pltpu.HBM((NPL * NP, PD), jnp.float32),
            pltpu.VMEM_SHARED((NP, PD), jnp.float32),
            pltpu.VMEM_SHARED((NP, PD), jnp.float32),
            pltpu.VMEM((CH, PD), jnp.float32),
            pltpu.VMEM((CH, PD), jnp.float32),
            pltpu.VMEM((CH, PD), jnp.float32),
            pltpu.VMEM((CH, PD), jnp.float32),
            pltpu.VMEM((QC, CH), jnp.int32),
            pltpu.VMEM((QC, CH), jnp.int32),
            pltpu.VMEM((QC, CH), jnp.float32),
            pltpu.VMEM((WCH, PD), jnp.float32),
            pltpu.VMEM((WCH, PD), jnp.float32),
        ] + [pltpu.SemaphoreType.DMA] * 15,
        compiler_params=pltpu.CompilerParams(
            needs_layout_passes=False, use_tc_tiling_on_sc=False),
    )
    return f(emb, cols2, rows2, vals2)


def kernel(user_embed, item_embed, edge_rows, edge_cols, edge_vals,
           batch, mess_dropout, edge_dropout):
    all_embed = jnp.concatenate([user_embed, item_embed], axis=0)
    emb = jnp.concatenate(
        [all_embed, jnp.zeros((NP - N, DIM), jnp.float32)], axis=0)
    pad = EPAD - E
    pr = jnp.arange(pad, dtype=jnp.int32) % N
    cols_p = jnp.concatenate([edge_cols.astype(jnp.int32), pr])
    rows_p = jnp.concatenate([edge_rows.astype(jnp.int32), pr])
    vals_p = jnp.concatenate([edge_vals, jnp.zeros((pad,), jnp.float32)])
    cols2 = cols_p.reshape(NCHUNK, CH)
    rows2 = rows_p.reshape(NCHUNK, CH)
    vals2 = vals_p.reshape(NCHUNK, CH)
    pooled = _run(emb, cols2, rows2, vals2)
    return (pooled[:N_USERS], pooled[N_USERS:N])
